# Initial kernel scaffold; baseline (speedup 1.0000x reference)
import jax
import jax.numpy as jnp
from jax.experimental import pallas as pl


def kernel(tokens, table):
    raise NotImplementedError("write your pallas kernel here")


# speedup vs baseline: 1.9673x; 1.9673x over previous
import functools
import jax
import jax.numpy as jnp
from jax import lax
from jax.experimental import pallas as pl
from jax.experimental.pallas import tpu as pltpu
from jax.experimental.pallas import tpu_sc as plsc

D = 128
L = 16
CHUNK = 128
EPS2 = (1e-12) ** 2 / 128.0


def _rsqrt(ssv):
    i = plsc.bitcast(ssv, jnp.int32)
    y = plsc.bitcast(jnp.int32(0x5F3759DF) - (i >> 1), jnp.float32)
    y = y * (jnp.float32(1.5) - jnp.float32(0.5) * ssv * y * y)
    y = y * (jnp.float32(1.5) - jnp.float32(0.5) * ssv * y * y)
    return y


def _normalize_chunk(rows_v, sq_v, inv_v):
    lane = lax.iota(jnp.int32, L)

    def group_body(g, carry):
        rb = g * L

        def row_a(i, carry):
            acc = None
            for j in range(D // L):
                x = rows_v[rb + i, pl.ds(j * L, L)]
                acc = x * x if acc is None else acc + x * x
            sq_v[i, pl.ds(0, L)] = acc
            return carry

        lax.fori_loop(0, L, row_a, None)

        t = None
        for j in range(L):
            v = plsc.load_gather(sq_v, [lane, jnp.full((L,), j, jnp.int32)])
            t = v if t is None else t + v
        inv_v[...] = _rsqrt(jnp.maximum(t, jnp.float32(EPS2)))

        def row_c(i, carry):
            iv = plsc.load_gather(inv_v, [jnp.full((L,), i, jnp.int32)])
            for j in range(D // L):
                rows_v[rb + i, pl.ds(j * L, L)] = (
                    rows_v[rb + i, pl.ds(j * L, L)] * iv
                )
            return carry

        lax.fori_loop(0, L, row_c, None)
        return carry

    lax.fori_loop(0, CHUNK // L, group_body, None)


def kernel(tokens, table):
    B = tokens.shape[0] * tokens.shape[1]
    info = plsc.get_sparse_core_info()
    NC, NS = info.num_cores, info.num_subcores
    NW = NC * NS
    b_per_w = B // NW
    n_chunks = b_per_w // CHUNK

    mesh = plsc.VectorSubcoreMesh(core_axis_name="c", subcore_axis_name="s")

    @functools.partial(
        pl.kernel,
        mesh=mesh,
        compiler_params=pltpu.CompilerParams(needs_layout_passes=False),
        out_type=jax.ShapeDtypeStruct((B, D), jnp.float32),
        scratch_types=[
            pltpu.VMEM((n_chunks, CHUNK), jnp.int32),
            pltpu.VMEM((CHUNK, D), jnp.float32),
            pltpu.VMEM((L, L + 1), jnp.float32),
            pltpu.VMEM((L,), jnp.float32),
            pltpu.SemaphoreType.DMA,
        ],
    )
    def sc_embed(idx_hbm, table_hbm, out_hbm, idx_v, rows_v, sq_v, inv_v, sem):
        wid = lax.axis_index("s") * NC + lax.axis_index("c")
        pltpu.sync_copy(idx_hbm.at[wid], idx_v)
        base = wid * b_per_w

        def chunk_body(g, _):
            pltpu.async_copy(table_hbm.at[idx_v.at[g]], rows_v, sem).wait()
            _normalize_chunk(rows_v, sq_v, inv_v)
            pltpu.sync_copy(rows_v, out_hbm.at[pl.ds(base + g * CHUNK, CHUNK), :])
            return _

        lax.fori_loop(0, n_chunks, chunk_body, None)

    idx = tokens.reshape(NW, n_chunks, CHUNK).astype(jnp.int32)
    out = sc_embed(idx, table)
    return out.reshape(tokens.shape[0], tokens.shape[1], D)


# speedup vs baseline: 2.4436x; 1.2421x over previous
import functools
import jax
import jax.numpy as jnp
from jax import lax
from jax.experimental import pallas as pl
from jax.experimental.pallas import tpu as pltpu
from jax.experimental.pallas import tpu_sc as plsc

D = 128
L = 16
CHUNK = 128
NBUF = 3
EPS2 = (1e-12) ** 2 / 128.0


def _rsqrt(ssv):
    i = plsc.bitcast(ssv, jnp.int32)
    y = plsc.bitcast(jnp.int32(0x5F3759DF) - (i >> 1), jnp.float32)
    y = y * (jnp.float32(1.5) - jnp.float32(0.5) * ssv * y * y)
    y = y * (jnp.float32(1.5) - jnp.float32(0.5) * ssv * y * y)
    return y


def _normalize_chunk(rows_v, sq_v, inv_v):
    lane = lax.iota(jnp.int32, L)

    def group_body(g, carry):
        rb = g * L

        def row_a(i, carry):
            acc = None
            for j in range(D // L):
                x = rows_v[rb + i, pl.ds(j * L, L)]
                acc = x * x if acc is None else acc + x * x
            sq_v[i, pl.ds(0, L)] = acc
            return carry

        lax.fori_loop(0, L, row_a, None)

        t = None
        for j in range(L):
            v = plsc.load_gather(sq_v, [lane, jnp.full((L,), j, jnp.int32)])
            t = v if t is None else t + v
        inv_v[...] = _rsqrt(jnp.maximum(t, jnp.float32(EPS2)))

        def row_c(i, carry):
            iv = plsc.load_gather(inv_v, [jnp.full((L,), i, jnp.int32)])
            for j in range(D // L):
                rows_v[rb + i, pl.ds(j * L, L)] = (
                    rows_v[rb + i, pl.ds(j * L, L)] * iv
                )
            return carry

        lax.fori_loop(0, L, row_c, None)
        return carry

    lax.fori_loop(0, CHUNK // L, group_body, None)


def kernel(tokens, table):
    B = tokens.shape[0] * tokens.shape[1]
    info = plsc.get_sparse_core_info()
    NC, NS = info.num_cores, info.num_subcores
    NW = NC * NS
    b_per_w = B // NW
    n_chunks = b_per_w // CHUNK

    mesh = plsc.VectorSubcoreMesh(core_axis_name="c", subcore_axis_name="s")

    @functools.partial(
        pl.kernel,
        mesh=mesh,
        compiler_params=pltpu.CompilerParams(needs_layout_passes=False),
        out_type=jax.ShapeDtypeStruct((B, D), jnp.float32),
        scratch_types=[
            pltpu.VMEM((n_chunks, CHUNK), jnp.int32),
            pltpu.VMEM((NBUF, CHUNK, D), jnp.float32),
            pltpu.VMEM((L, L + 1), jnp.float32),
            pltpu.VMEM((L,), jnp.float32),
            pltpu.SemaphoreType.DMA((NBUF,)),
            pltpu.SemaphoreType.DMA((NBUF,)),
        ],
    )
    def sc_embed(idx_hbm, table_hbm, out_hbm, idx_v, rows_v, sq_v, inv_v,
                 sem_in, sem_out):
        wid = lax.axis_index("s") * NC + lax.axis_index("c")
        pltpu.sync_copy(idx_hbm.at[wid], idx_v)
        base = wid * b_per_w

        def gather_copy(g, b):
            return pltpu.make_async_copy(
                table_hbm.at[idx_v.at[g]], rows_v.at[b], sem_in.at[b])

        def out_copy(g, b):
            return pltpu.make_async_copy(
                rows_v.at[b],
                out_hbm.at[pl.ds(base + g * CHUNK, CHUNK), :],
                sem_out.at[b])

        gather_copy(0, 0).start()
        gather_copy(1, 1).start()

        def chunk_body(g, _):
            b = lax.rem(g, NBUF)
            gather_copy(g, b).wait()
            _normalize_chunk(rows_v.at[b], sq_v, inv_v)
            out_copy(g, b).start()

            @pl.when(g + 2 < n_chunks)
            def _prefetch():
                nb = lax.rem(g + 2, NBUF)

                @pl.when(g >= 1)
                def _drain():
                    out_copy(g - 1, nb).wait()

                gather_copy(g + 2, nb).start()

            return _

        lax.fori_loop(0, n_chunks, chunk_body, None)
        for g in range(n_chunks - 3, n_chunks):
            out_copy(g, g % NBUF).wait()

    idx = tokens.reshape(NW, n_chunks, CHUNK).astype(jnp.int32)
    out = sc_embed(idx, table)
    return out.reshape(tokens.shape[0], tokens.shape[1], D)


# speedup vs baseline: 2.5217x; 1.0320x over previous
import functools
import jax
import jax.numpy as jnp
from jax import lax
from jax.experimental import pallas as pl
from jax.experimental.pallas import tpu as pltpu
from jax.experimental.pallas import tpu_sc as plsc

D = 128
L = 16
CHUNK = 128
NBUF = 3
EPS2 = (1e-12) ** 2 / 128.0


def _rsqrt(ssv):
    i = plsc.bitcast(ssv, jnp.int32)
    y = plsc.bitcast(jnp.int32(0x5F3759DF) - (i >> 1), jnp.float32)
    y = y * (jnp.float32(1.5) - jnp.float32(0.5) * ssv * y * y)
    y = y * (jnp.float32(1.5) - jnp.float32(0.5) * ssv * y * y)
    return y


def _normalize_chunk(rows_v, sq_v, inv_v):
    lane = lax.iota(jnp.int32, L)

    def group_body(g, carry):
        rb = g * L

        def rows_a(i4, carry):
            for k in range(4):
                i = i4 * 4 + k
                acc = None
                for j in range(D // L):
                    x = rows_v[rb + i, pl.ds(j * L, L)]
                    acc = x * x if acc is None else acc + x * x
                sq_v[i, pl.ds(0, L)] = acc
            return carry

        lax.fori_loop(0, L // 4, rows_a, None)

        t = None
        for j in range(L):
            v = plsc.load_gather(sq_v, [lane, jnp.full((L,), j, jnp.int32)])
            t = v if t is None else t + v
        inv_v[...] = _rsqrt(jnp.maximum(t, jnp.float32(EPS2)))

        def rows_c(i4, carry):
            for k in range(4):
                i = i4 * 4 + k
                iv = plsc.load_gather(inv_v, [jnp.full((L,), i, jnp.int32)])
                for j in range(D // L):
                    rows_v[rb + i, pl.ds(j * L, L)] = (
                        rows_v[rb + i, pl.ds(j * L, L)] * iv
                    )
            return carry

        lax.fori_loop(0, L // 4, rows_c, None)
        return carry

    lax.fori_loop(0, CHUNK // L, group_body, None)


def kernel(tokens, table):
    B = tokens.shape[0] * tokens.shape[1]
    info = plsc.get_sparse_core_info()
    NC, NS = info.num_cores, info.num_subcores
    NW = NC * NS
    b_per_w = B // NW
    n_chunks = b_per_w // CHUNK

    mesh = plsc.VectorSubcoreMesh(core_axis_name="c", subcore_axis_name="s")

    @functools.partial(
        pl.kernel,
        mesh=mesh,
        compiler_params=pltpu.CompilerParams(needs_layout_passes=False),
        out_type=jax.ShapeDtypeStruct((B, D), jnp.float32),
        scratch_types=[
            pltpu.VMEM((n_chunks, CHUNK), jnp.int32),
            pltpu.VMEM((NBUF, CHUNK, D), jnp.float32),
            pltpu.VMEM((L, L + 1), jnp.float32),
            pltpu.VMEM((L,), jnp.float32),
            pltpu.SemaphoreType.DMA((NBUF,)),
            pltpu.SemaphoreType.DMA((NBUF,)),
        ],
    )
    def sc_embed(idx_hbm, table_hbm, out_hbm, idx_v, rows_v, sq_v, inv_v,
                 sem_in, sem_out):
        wid = lax.axis_index("s") * NC + lax.axis_index("c")
        pltpu.sync_copy(idx_hbm.at[wid], idx_v)
        base = wid * b_per_w

        def gather_copy(g, b):
            return pltpu.make_async_copy(
                table_hbm.at[idx_v.at[g]], rows_v.at[b], sem_in.at[b])

        def out_copy(g, b):
            return pltpu.make_async_copy(
                rows_v.at[b],
                out_hbm.at[pl.ds(base + g * CHUNK, CHUNK), :],
                sem_out.at[b])

        gather_copy(0, 0).start()
        gather_copy(1, 1).start()

        def chunk_body(g, _):
            b = lax.rem(g, NBUF)
            gather_copy(g, b).wait()
            _normalize_chunk(rows_v.at[b], sq_v, inv_v)
            out_copy(g, b).start()

            @pl.when(g + 2 < n_chunks)
            def _prefetch():
                nb = lax.rem(g + 2, NBUF)

                @pl.when(g >= 1)
                def _drain():
                    out_copy(g - 1, nb).wait()

                gather_copy(g + 2, nb).start()

            return _

        lax.fori_loop(0, n_chunks, chunk_body, None)
        for g in range(n_chunks - 3, n_chunks):
            out_copy(g, g % NBUF).wait()

    idx = tokens.reshape(NW, n_chunks, CHUNK).astype(jnp.int32)
    out = sc_embed(idx, table)
    return out.reshape(tokens.shape[0], tokens.shape[1], D)
